# baseline (device time: 69027 ns/iter reference)
import jax
import jax.numpy as jnp
from jax import lax
from jax.experimental import pallas as pl
from jax.experimental.pallas import tpu as pltpu

N_DEV = 8
N_TOK = 1024
D_IN = 512
D_OUT = 1024
E_LOCAL = 4
CAPACITY = 25
M_PER = N_TOK // N_DEV


def kernel(x, router_W, route_idx, expert_W):
    del router_W

    def body(x_ref, ridx_ref, ew_ref, out_ref,
             acc_ref, snd_ref, comm_ref, send_sems, recv_sems):
        my = lax.axis_index("i")
        left = lax.rem(my + N_DEV - 1, N_DEV)
        right = lax.rem(my + 1, N_DEV)

        barrier_sem = pltpu.get_barrier_semaphore()
        for nbr in [left, right]:
            pl.semaphore_signal(
                barrier_sem, inc=1,
                device_id=(nbr,), device_id_type=pl.DeviceIdType.MESH,
            )
        pl.semaphore_wait(barrier_sem, 2)

        local_e = my * E_LOCAL + lax.broadcasted_iota(jnp.int32, (N_TOK, E_LOCAL), 1)
        onehot = (ridx_ref[:, :] == local_e).astype(jnp.float32)
        row = lax.broadcasted_iota(jnp.int32, (N_TOK, N_TOK), 0)
        col = lax.broadcasted_iota(jnp.int32, (N_TOK, N_TOK), 1)
        tril = (col <= row).astype(jnp.float32)
        incl = jnp.dot(tril, onehot, preferred_element_type=jnp.float32)
        keep = onehot * (incl <= CAPACITY).astype(jnp.float32)

        xf = x_ref[:, :]
        acc = jnp.zeros((N_TOK, D_OUT), jnp.float32)
        for j in range(E_LOCAL):
            xm = (xf * keep[:, j:j + 1]).astype(jnp.bfloat16)
            wj = ew_ref[j, :, :].astype(jnp.bfloat16)
            acc = acc + jnp.dot(xm, wj, preferred_element_type=jnp.float32)
        acc_ref[:, :] = acc

        for t in range(N_DEV - 1):
            c_send = lax.rem(my - 1 - t + 2 * N_DEV, N_DEV)
            if t == 0:
                snd_ref[:, :] = acc_ref[pl.ds(c_send * M_PER, M_PER), :]
            else:
                snd_ref[:, :] = (comm_ref[t - 1, :, :]
                                 + acc_ref[pl.ds(c_send * M_PER, M_PER), :])
            rdma = pltpu.make_async_remote_copy(
                src_ref=snd_ref,
                dst_ref=comm_ref.at[t],
                send_sem=send_sems.at[t],
                recv_sem=recv_sems.at[t],
                device_id=(right,),
                device_id_type=pl.DeviceIdType.MESH,
            )
            rdma.start()
            rdma.wait()

        out_ref[:, :] = (comm_ref[N_DEV - 2, :, :]
                         + acc_ref[pl.ds(my * M_PER, M_PER), :])

    return pl.pallas_call(
        body,
        out_shape=jax.ShapeDtypeStruct((M_PER, D_OUT), jnp.float32),
        in_specs=[
            pl.BlockSpec(memory_space=pltpu.VMEM),
            pl.BlockSpec(memory_space=pltpu.VMEM),
            pl.BlockSpec(memory_space=pltpu.VMEM),
        ],
        out_specs=pl.BlockSpec(memory_space=pltpu.VMEM),
        scratch_shapes=[
            pltpu.VMEM((N_TOK, D_OUT), jnp.float32),
            pltpu.VMEM((M_PER, D_OUT), jnp.float32),
            pltpu.VMEM((N_DEV - 1, M_PER, D_OUT), jnp.float32),
            pltpu.SemaphoreType.DMA((N_DEV - 1,)),
            pltpu.SemaphoreType.DMA((N_DEV - 1,)),
        ],
        compiler_params=pltpu.CompilerParams(collective_id=0),
    )(x, route_idx, expert_W)


# device time: 49252 ns/iter; 1.4015x vs baseline; 1.4015x over previous
import jax
import jax.numpy as jnp
from jax import lax
from jax.experimental import pallas as pl
from jax.experimental.pallas import tpu as pltpu

N_DEV = 8
N_TOK = 1024
D_IN = 512
D_OUT = 1024
E_LOCAL = 4
CAPACITY = 25
M_PER = N_TOK // N_DEV


def kernel(x, router_W, route_idx, expert_W):
    del router_W

    def body(x_ref, ridx_ref, ew_ref, out_ref,
             acc_ref, snd_ref, comm_ref, send_sems, recv_sems):
        my = lax.axis_index("i")
        left = lax.rem(my + N_DEV - 1, N_DEV)
        right = lax.rem(my + 1, N_DEV)

        barrier_sem = pltpu.get_barrier_semaphore()
        for nbr in [left, right]:
            pl.semaphore_signal(
                barrier_sem, inc=1,
                device_id=(nbr,), device_id_type=pl.DeviceIdType.MESH,
            )
        pl.semaphore_wait(barrier_sem, 2)

        local_e = my * E_LOCAL + lax.broadcasted_iota(jnp.int32, (N_TOK, E_LOCAL), 1)
        onehot = (ridx_ref[:, :] == local_e).astype(jnp.float32)
        row = lax.broadcasted_iota(jnp.int32, (N_TOK, N_TOK), 0)
        col = lax.broadcasted_iota(jnp.int32, (N_TOK, N_TOK), 1)
        tril = (col <= row).astype(jnp.float32)
        incl = jnp.dot(tril, onehot, preferred_element_type=jnp.float32)
        keep = onehot * (incl <= CAPACITY).astype(jnp.float32)

        xf = x_ref[:, :]
        acc = jnp.zeros((N_TOK, D_OUT), jnp.float32)
        for j in range(E_LOCAL):
            xm = (xf * keep[:, j:j + 1]).astype(jnp.bfloat16)
            wj = ew_ref[j, :, :].astype(jnp.bfloat16)
            acc = acc + jnp.dot(xm, wj, preferred_element_type=jnp.float32)
        acc_ref[:, :] = acc

        for t in range(N_DEV - 1):
            c_send = lax.rem(my - 1 - t + 2 * N_DEV, N_DEV)
            if t == 0:
                snd_ref[:, :] = acc_ref[pl.ds(c_send * M_PER, M_PER), :].astype(jnp.bfloat16)
            else:
                snd_ref[:, :] = (comm_ref[t - 1, :, :].astype(jnp.float32)
                                 + acc_ref[pl.ds(c_send * M_PER, M_PER), :]
                                 ).astype(jnp.bfloat16)
            rdma = pltpu.make_async_remote_copy(
                src_ref=snd_ref,
                dst_ref=comm_ref.at[t],
                send_sem=send_sems.at[t],
                recv_sem=recv_sems.at[t],
                device_id=(right,),
                device_id_type=pl.DeviceIdType.MESH,
            )
            rdma.start()
            rdma.wait()

        out_ref[:, :] = (comm_ref[N_DEV - 2, :, :].astype(jnp.float32)
                         + acc_ref[pl.ds(my * M_PER, M_PER), :])

    return pl.pallas_call(
        body,
        out_shape=jax.ShapeDtypeStruct((M_PER, D_OUT), jnp.float32),
        in_specs=[
            pl.BlockSpec(memory_space=pltpu.VMEM),
            pl.BlockSpec(memory_space=pltpu.VMEM),
            pl.BlockSpec(memory_space=pltpu.VMEM),
        ],
        out_specs=pl.BlockSpec(memory_space=pltpu.VMEM),
        scratch_shapes=[
            pltpu.VMEM((N_TOK, D_OUT), jnp.float32),
            pltpu.VMEM((M_PER, D_OUT), jnp.bfloat16),
            pltpu.VMEM((N_DEV - 1, M_PER, D_OUT), jnp.bfloat16),
            pltpu.SemaphoreType.DMA((N_DEV - 1,)),
            pltpu.SemaphoreType.DMA((N_DEV - 1,)),
        ],
        compiler_params=pltpu.CompilerParams(collective_id=0),
    )(x, route_idx, expert_W)


# device time: 33652 ns/iter; 2.0512x vs baseline; 1.4636x over previous
import jax
import jax.numpy as jnp
from jax import lax
from jax.experimental import pallas as pl
from jax.experimental.pallas import tpu as pltpu

N_DEV = 8
N_TOK = 1024
D_IN = 512
D_OUT = 1024
E_LOCAL = 4
CAPACITY = 25
M_PER = N_TOK // N_DEV


def kernel(x, router_W, route_idx, expert_W):
    del router_W

    def body(x_ref, ridx_ref, ew_ref, out_ref,
             acc_ref, comm_ref, send_sems, recv_sems):
        my = lax.axis_index("i")

        barrier_sem = pltpu.get_barrier_semaphore()
        for k in range(1, N_DEV):
            peer = lax.rem(my + k, N_DEV)
            pl.semaphore_signal(
                barrier_sem, inc=1,
                device_id=(peer,), device_id_type=pl.DeviceIdType.MESH,
            )
        pl.semaphore_wait(barrier_sem, N_DEV - 1)

        local_e = my * E_LOCAL + lax.broadcasted_iota(jnp.int32, (N_TOK, E_LOCAL), 1)
        onehot = (ridx_ref[:, :] == local_e).astype(jnp.float32)
        row = lax.broadcasted_iota(jnp.int32, (N_TOK, N_TOK), 0)
        col = lax.broadcasted_iota(jnp.int32, (N_TOK, N_TOK), 1)
        tril = (col <= row).astype(jnp.float32)
        incl = jnp.dot(tril, onehot, preferred_element_type=jnp.float32)
        keep = onehot * (incl <= CAPACITY).astype(jnp.float32)

        xf = x_ref[:, :]
        acc = jnp.zeros((N_TOK, D_OUT), jnp.float32)
        for j in range(E_LOCAL):
            xm = (xf * keep[:, j:j + 1]).astype(jnp.bfloat16)
            wj = ew_ref[j, :, :].astype(jnp.bfloat16)
            acc = acc + jnp.dot(xm, wj, preferred_element_type=jnp.float32)
        acc_ref[:, :] = acc.astype(jnp.bfloat16)

        rdmas = []
        for k in range(1, N_DEV):
            dst = lax.rem(my + k, N_DEV)
            rdma = pltpu.make_async_remote_copy(
                src_ref=acc_ref.at[pl.ds(dst * M_PER, M_PER), :],
                dst_ref=comm_ref.at[k - 1],
                send_sem=send_sems.at[k - 1],
                recv_sem=recv_sems.at[k - 1],
                device_id=(dst,),
                device_id_type=pl.DeviceIdType.MESH,
            )
            rdma.start()
            rdmas.append(rdma)

        total = acc_ref[pl.ds(my * M_PER, M_PER), :].astype(jnp.float32)
        for k in range(1, N_DEV):
            rdmas[k - 1].wait()
            total = total + comm_ref[k - 1, :, :].astype(jnp.float32)
        out_ref[:, :] = total

    return pl.pallas_call(
        body,
        out_shape=jax.ShapeDtypeStruct((M_PER, D_OUT), jnp.float32),
        in_specs=[
            pl.BlockSpec(memory_space=pltpu.VMEM),
            pl.BlockSpec(memory_space=pltpu.VMEM),
            pl.BlockSpec(memory_space=pltpu.VMEM),
        ],
        out_specs=pl.BlockSpec(memory_space=pltpu.VMEM),
        scratch_shapes=[
            pltpu.VMEM((N_TOK, D_OUT), jnp.bfloat16),
            pltpu.VMEM((N_DEV - 1, M_PER, D_OUT), jnp.bfloat16),
            pltpu.SemaphoreType.DMA((N_DEV - 1,)),
            pltpu.SemaphoreType.DMA((N_DEV - 1,)),
        ],
        compiler_params=pltpu.CompilerParams(collective_id=0),
    )(x, route_idx, expert_W)


# device time: 29948 ns/iter; 2.3049x vs baseline; 1.1237x over previous
import jax
import jax.numpy as jnp
from jax import lax
from jax.experimental import pallas as pl
from jax.experimental.pallas import tpu as pltpu

N_DEV = 8
N_TOK = 1024
D_IN = 512
D_OUT = 1024
E_LOCAL = 4
CAPACITY = 25
M_PER = N_TOK // N_DEV


def kernel(x, router_W, route_idx, expert_W):
    del router_W

    def body(x_ref, ridx_ref, ew_ref, out_ref,
             keep_ref, acc_ref, comm_ref, send_sems, recv_sems):
        my = lax.axis_index("i")

        barrier_sem = pltpu.get_barrier_semaphore()
        for k in range(1, N_DEV):
            peer = lax.rem(my + k, N_DEV)
            pl.semaphore_signal(
                barrier_sem, inc=1,
                device_id=(peer,), device_id_type=pl.DeviceIdType.MESH,
            )
        pl.semaphore_wait(barrier_sem, N_DEV - 1)

        local_e = my * E_LOCAL + lax.broadcasted_iota(jnp.int32, (N_TOK, E_LOCAL), 1)
        onehot = (ridx_ref[:, :] == local_e).astype(jnp.float32)
        row = lax.broadcasted_iota(jnp.int32, (N_TOK, N_TOK), 0)
        col = lax.broadcasted_iota(jnp.int32, (N_TOK, N_TOK), 1)
        tril = (col <= row).astype(jnp.float32)
        incl = jnp.dot(tril, onehot, preferred_element_type=jnp.float32)
        keep = onehot * (incl <= CAPACITY).astype(jnp.float32)

        keep_ref[:, :] = keep

        wq = ew_ref[:, :, :].astype(jnp.bfloat16)
        rdmas = []
        for k in range(1, N_DEV):
            dst = lax.rem(my + k, N_DEV)
            r0 = dst * M_PER
            xc = x_ref[pl.ds(r0, M_PER), :]
            kc = keep_ref[pl.ds(r0, M_PER), :]
            acc = jnp.zeros((M_PER, D_OUT), jnp.float32)
            for j in range(E_LOCAL):
                xm = (xc * kc[:, j:j + 1]).astype(jnp.bfloat16)
                acc = acc + jnp.dot(xm, wq[j], preferred_element_type=jnp.float32)
            acc_ref[pl.ds(r0, M_PER), :] = acc.astype(jnp.bfloat16)
            rdma = pltpu.make_async_remote_copy(
                src_ref=acc_ref.at[pl.ds(r0, M_PER), :],
                dst_ref=comm_ref.at[k - 1],
                send_sem=send_sems.at[k - 1],
                recv_sem=recv_sems.at[k - 1],
                device_id=(dst,),
                device_id_type=pl.DeviceIdType.MESH,
            )
            rdma.start()
            rdmas.append(rdma)

        r0 = my * M_PER
        xc = x_ref[pl.ds(r0, M_PER), :]
        kc = keep_ref[pl.ds(r0, M_PER), :]
        acc = jnp.zeros((M_PER, D_OUT), jnp.float32)
        for j in range(E_LOCAL):
            xm = (xc * kc[:, j:j + 1]).astype(jnp.bfloat16)
            acc = acc + jnp.dot(xm, wq[j], preferred_element_type=jnp.float32)
        acc_ref[pl.ds(r0, M_PER), :] = acc.astype(jnp.bfloat16)

        total = acc_ref[pl.ds(my * M_PER, M_PER), :].astype(jnp.float32)
        for k in range(1, N_DEV):
            rdmas[k - 1].wait()
            total = total + comm_ref[k - 1, :, :].astype(jnp.float32)
        out_ref[:, :] = total

    return pl.pallas_call(
        body,
        out_shape=jax.ShapeDtypeStruct((M_PER, D_OUT), jnp.float32),
        in_specs=[
            pl.BlockSpec(memory_space=pltpu.VMEM),
            pl.BlockSpec(memory_space=pltpu.VMEM),
            pl.BlockSpec(memory_space=pltpu.VMEM),
        ],
        out_specs=pl.BlockSpec(memory_space=pltpu.VMEM),
        scratch_shapes=[
            pltpu.VMEM((N_TOK, E_LOCAL), jnp.float32),
            pltpu.VMEM((N_TOK, D_OUT), jnp.bfloat16),
            pltpu.VMEM((N_DEV - 1, M_PER, D_OUT), jnp.bfloat16),
            pltpu.SemaphoreType.DMA((N_DEV - 1,)),
            pltpu.SemaphoreType.DMA((N_DEV - 1,)),
        ],
        compiler_params=pltpu.CompilerParams(collective_id=0),
    )(x, route_idx, expert_W)
